# Initial kernel scaffold; baseline (speedup 1.0000x reference)
import jax
import jax.numpy as jnp
from jax.experimental import pallas as pl


def kernel(z_e, codebook):
    raise NotImplementedError("write your pallas kernel here")


# speedup vs baseline: 1.1451x; 1.1451x over previous
import functools

import jax
import jax.numpy as jnp
from jax.experimental import pallas as pl

NUM_CODES = 8192
DIM = 64
COMMITMENT_COST = 0.25
TM = 512


def _argmin_body(z_ref, cbt_ref, sumz_ref, sume_ref, idx_ref, mind_ref):
    z = z_ref[...]
    cbt = cbt_ref[...]
    dots = jnp.dot(z, cbt, preferred_element_type=jnp.float32)
    d = (sumz_ref[...] - 2.0 * dots) + sume_ref[...]
    minval = jnp.min(d, axis=1, keepdims=True)
    iota = jax.lax.broadcasted_iota(jnp.int32, d.shape, 1)
    idx = jnp.min(jnp.where(d == minval, iota, NUM_CODES), axis=1)
    idx_ref[...] = idx
    mind_ref[...] = minval[:, 0]


@functools.partial(jax.jit, static_argnames=("interpret",))
def _vq(z_e, codebook, interpret=False):
    B, C, H, W = z_e.shape
    N = B * H * W
    z = jnp.transpose(z_e, (0, 2, 3, 1))
    z_flat = z.reshape(-1, C)
    sumz = jnp.sum(z_flat ** 2, axis=1, keepdims=True)
    sume = jnp.sum(codebook ** 2, axis=1).reshape(1, -1)
    cbt = codebook.T

    idx, mind = pl.pallas_call(
        _argmin_body,
        grid=(N // TM,),
        in_specs=[
            pl.BlockSpec((TM, DIM), lambda i: (i, 0)),
            pl.BlockSpec((DIM, NUM_CODES), lambda i: (0, 0)),
            pl.BlockSpec((TM, 1), lambda i: (i, 0)),
            pl.BlockSpec((1, NUM_CODES), lambda i: (0, 0)),
        ],
        out_specs=[
            pl.BlockSpec((TM,), lambda i: (i,)),
            pl.BlockSpec((TM,), lambda i: (i,)),
        ],
        out_shape=[
            jax.ShapeDtypeStruct((N,), jnp.int32),
            jax.ShapeDtypeStruct((N,), jnp.float32),
        ],
        interpret=interpret,
    )(z_flat, cbt, sumz, sume)

    z_q_flat = jnp.take(codebook, idx, axis=0)
    z_q = z_q_flat.reshape(z.shape)
    loss = COMMITMENT_COST * (jnp.sum(mind) / (N * C))
    z_q_ste = z + jax.lax.stop_gradient(z_q - z)
    z_q_ste = jnp.transpose(z_q_ste, (0, 3, 1, 2))
    return z_q_ste, loss, idx.reshape(B, H, W)


def kernel(z_e, codebook):
    return _vq(z_e, codebook)


# speedup vs baseline: 1.1591x; 1.0122x over previous
import functools

import jax
import jax.numpy as jnp
from jax.experimental import pallas as pl

NUM_CODES = 8192
DIM = 64
COMMITMENT_COST = 0.25
TN = 1024


def _argmin_body(z_ref, cb2_ref, sumz_ref, sume_ref, idx_ref, mind_ref):
    z = z_ref[0]
    cb2 = cb2_ref[...]
    dots2 = jnp.dot(cb2, z, preferred_element_type=jnp.float32)
    d = (sumz_ref[0] - dots2) + sume_ref[...]
    minval = jnp.min(d, axis=0, keepdims=True)
    iota = jax.lax.broadcasted_iota(jnp.int32, d.shape, 0)
    idx = jnp.min(jnp.where(d == minval, iota, NUM_CODES), axis=0)
    idx_ref[0, 0] = idx
    mind_ref[0, 0] = minval[0]


@functools.partial(jax.jit, static_argnames=("interpret",))
def _vq(z_e, codebook, interpret=False):
    B, C, H, W = z_e.shape
    N = B * H * W
    HW = H * W
    z2 = z_e.reshape(B, C, HW)
    z_flat = jnp.transpose(z_e, (0, 2, 3, 1)).reshape(-1, C)
    sumz = jnp.sum(z_flat ** 2, axis=1).reshape(B, 1, HW)
    sume = jnp.sum(codebook ** 2, axis=1).reshape(-1, 1)
    cb2 = codebook * 2.0

    idx, mind = pl.pallas_call(
        _argmin_body,
        grid=(B * HW // TN,),
        in_specs=[
            pl.BlockSpec((1, DIM, TN), lambda i: (i, 0, 0)),
            pl.BlockSpec((NUM_CODES, DIM), lambda i: (0, 0)),
            pl.BlockSpec((1, 1, TN), lambda i: (i, 0, 0)),
            pl.BlockSpec((NUM_CODES, 1), lambda i: (0, 0)),
        ],
        out_specs=[
            pl.BlockSpec((1, 1, TN), lambda i: (i, 0, 0)),
            pl.BlockSpec((1, 1, TN), lambda i: (i, 0, 0)),
        ],
        out_shape=[
            jax.ShapeDtypeStruct((N // TN, 1, TN), jnp.int32),
            jax.ShapeDtypeStruct((N // TN, 1, TN), jnp.float32),
        ],
        interpret=interpret,
    )(z2, cb2, sumz, sume)

    idx = idx.reshape(N)
    z_q_flat = jnp.take(codebook, idx, axis=0)
    z_q_bchw = jnp.transpose(z_q_flat.reshape(B, H, W, C), (0, 3, 1, 2))
    loss = COMMITMENT_COST * (jnp.sum(mind) / (N * C))
    z_q_ste = z_e + jax.lax.stop_gradient(z_q_bchw - z_e)
    return z_q_ste, loss, idx.reshape(B, H, W)


def kernel(z_e, codebook):
    return _vq(z_e, codebook)
